# baseline (speedup 1.0000x reference)
import functools

import jax
import jax.numpy as jnp
from jax import lax
from jax.experimental import pallas as pl
from jax.experimental.pallas import tpu as pltpu

NUM_NODES = 1000000
EMBED_DIM = 64
BATCH = 16384
_NSEM = 16


def _body(idx_smem, table_hbm, out_hbm, rows_v, sems):
    def fire(i, carry):
        r = idx_smem[i]
        pltpu.make_async_copy(
            table_hbm.at[pl.ds(r, 1)],
            rows_v.at[pl.ds(i, 1)],
            sems.at[lax.rem(i, _NSEM)],
        ).start()
        return carry

    lax.fori_loop(0, BATCH, fire, 0)
    for q in range(_NSEM):
        pltpu.make_async_copy(
            table_hbm.at[pl.ds(0, BATCH // _NSEM)],
            rows_v.at[pl.ds(0, BATCH // _NSEM)],
            sems.at[q],
        ).wait()
    pltpu.make_async_copy(rows_v, out_hbm, sems.at[0]).start()
    pltpu.make_async_copy(rows_v, out_hbm, sems.at[0]).wait()


def kernel(indices, weight):
    idx = indices.astype(jnp.int32)
    return pl.pallas_call(
        _body,
        grid_spec=pltpu.PrefetchScalarGridSpec(
            num_scalar_prefetch=1,
            grid=(1,),
            in_specs=[pl.BlockSpec(memory_space=pl.ANY)],
            out_specs=pl.BlockSpec(memory_space=pl.ANY),
            scratch_shapes=[
                pltpu.VMEM((BATCH, EMBED_DIM), jnp.float32),
                pltpu.SemaphoreType.DMA((_NSEM,)),
            ],
        ),
        out_shape=jax.ShapeDtypeStruct((BATCH, EMBED_DIM), jnp.float32),
    )(idx, weight)
